# baseline (speedup 1.0000x reference)
import jax
import jax.numpy as jnp
from jax.experimental import pallas as pl


def _copy_body(x_ref, o_ref):
    o_ref[...] = x_ref[...]


def _gcn(x, src, dst, dinv, W, b, n):
    h = x @ W
    norm = dinv[src] * dinv[dst]
    msg = h[src] * norm[:, None]
    agg = jax.ops.segment_sum(msg, dst, num_segments=n)
    return agg + b


def kernel(x, edge_index, batch, Wg0, bg0, Wl0, bl0, Wg1, bg1, Wl1, bl1, Wlast, blast):
    n = x.shape[0]
    src = edge_index[0]
    dst = edge_index[1]
    loop = jnp.arange(n, dtype=src.dtype)
    src = jnp.concatenate([src, loop])
    dst = jnp.concatenate([dst, loop])
    deg = jnp.zeros((n,), x.dtype).at[dst].add(1.0)
    dinv = jnp.where(deg > 0, 1.0 / jnp.sqrt(deg), 0.0)
    h = x
    for Wg, bg, Wl, bl in [(Wg0, bg0, Wl0, bl0), (Wg1, bg1, Wl1, bl1)]:
        h = _gcn(h, src, dst, dinv, Wg, bg, n) + (h @ Wl + bl) * 0.5
        h = jax.nn.relu(h)
    h = _gcn(h, src, dst, dinv, Wlast, blast, n)
    order = jnp.argsort(-h[:, -1])
    h = jnp.take(h, order[:1000], axis=0)
    out = pl.pallas_call(
        _copy_body,
        out_shape=jax.ShapeDtypeStruct(h.shape, h.dtype),
    )(h)
    return out.reshape(1000, 128)


# speedup vs baseline: 3.9003x; 3.9003x over previous
import functools

import jax
import jax.numpy as jnp
from jax import lax
from jax.experimental import pallas as pl
from jax.experimental.pallas import tpu as pltpu
from jax.experimental.pallas import tpu_sc as plsc

N = 10000
D = 128
POOL = 1000
E = 320000

NC = 2
NS = 16
NW = NC * NS
CHUNK = 128
EW = 10240
NCH = EW // CHUNK
EPAD = EW * NW

NPAD = 10240
RANGE = NPAD // NW
RPS = NPAD // NS
NPADP = NPAD + 512
CAP = 12288
CAPC = CAP // CHUNK
SCANC = 1600
NSCAN = E // SCANC
PCH = 128
RB = 512
NBLK = NPAD // RB
JC = 512
IB = 128
TPOOL = 1024

_f32 = jnp.float32
_i32 = jnp.int32

_mesh = plsc.VectorSubcoreMesh(core_axis_name="c", subcore_axis_name="s",
                               num_cores=NC, num_subcores=NS)


def _scalar(x):
    return x[0] if getattr(x, "ndim", 0) else x


@functools.partial(
    pl.kernel,
    out_type=jax.ShapeDtypeStruct((NC, NPAD), _f32),
    mesh=_mesh,
    scratch_types=[
        pltpu.VMEM_SHARED((NPAD,), _f32),
        pltpu.VMEM((NCH, CHUNK), _i32),
        pltpu.VMEM((CHUNK,), _f32),
    ],
)
def _sc_degree(dst_hbm, zrow_hbm, out_hbm, deg_s, idx_v, ones_v):
    c = lax.axis_index("c")
    s = lax.axis_index("s")
    w = s * NC + c
    pltpu.sync_copy(zrow_hbm, deg_s.at[pl.ds(s * RPS, RPS)])
    for k in range(CHUNK // 16):
        ones_v[pl.ds(k * 16, 16)] = jnp.ones((16,), _f32)
    pltpu.sync_copy(dst_hbm.at[w], idx_v)
    plsc.subcore_barrier()

    @pl.loop(0, NCH)
    def _chunks(j):
        pltpu.sync_copy(ones_v, deg_s.at[idx_v.at[j]], add=True)

    plsc.subcore_barrier()
    pltpu.sync_copy(deg_s.at[pl.ds(s * RPS, RPS)],
                    out_hbm.at[c, pl.ds(s * RPS, RPS)])


@functools.partial(
    pl.kernel,
    out_type=(jax.ShapeDtypeStruct((NW, CAP), _i32),
              jax.ShapeDtypeStruct((NW, 16), _i32)),
    mesh=_mesh,
    compiler_params=pltpu.CompilerParams(needs_layout_passes=False),
    scratch_types=[
        pltpu.VMEM((CAP,), _i32),
        pltpu.VMEM((SCANC,), _i32),
        pltpu.VMEM((SCANC,), _i32),
        pltpu.VMEM((16,), _i32),
    ],
)
def _sc_prep(src_hbm, dst_hbm, pk_hbm, cnt_hbm, pk_v, sbuf, dbuf, cnt_v):
    c = lax.axis_index("c")
    s = lax.axis_index("s")
    w = s * NC + c
    lo = w * RANGE
    cnt_v[pl.ds(0, 16)] = jnp.zeros((16,), _i32)

    @pl.loop(0, CAP // 16)
    def _fill(k):
        pk_v[pl.ds(k * 16, 16)] = jnp.full((16,), RANGE, _i32)

    lane = lax.iota(_i32, 16)

    def _vec(k, off):
        sv = sbuf[pl.ds(k * 16, 16)]
        dv = dbuf[pl.ds(k * 16, 16)]
        m = (dv >= lo) & (dv < lo + RANGE)
        key = jnp.where(m, lane, lane + 64)
        packed = jnp.where(m, sv * 512 + (dv - lo), RANGE)
        _, sval = plsc.sort_key_val(key, packed)

        @pl.when(off <= CAP - 16)
        def _store():
            pk_v[pl.ds(off, 16)] = sval

        plsc.addupdate_scatter(cnt_v, [jnp.zeros((16,), _i32)], m.astype(_i32))
        return cnt_v[pl.ds(0, 16)][0]

    def _chunk(ci, off):
        pltpu.sync_copy(src_hbm.at[pl.ds(ci * SCANC, SCANC)], sbuf)
        pltpu.sync_copy(dst_hbm.at[pl.ds(ci * SCANC, SCANC)], dbuf)
        return pl.loop(0, SCANC // 16, init_carry=off)(_vec)

    off = pl.loop(0, NSCAN, init_carry=jnp.int32(0))(_chunk)
    cnt_v[...] = jnp.zeros((16,), _i32) + off
    pltpu.sync_copy(pk_v, pk_hbm.at[w])
    pltpu.sync_copy(cnt_v, cnt_hbm.at[w])


@functools.partial(
    pl.kernel,
    out_type=jax.ShapeDtypeStruct((NPAD * D,), _f32),
    mesh=_mesh,
    compiler_params=pltpu.CompilerParams(needs_layout_passes=False),
    scratch_types=[
        pltpu.VMEM(((RANGE + 1) * D,), _f32),
        pltpu.VMEM((CAP,), _i32),
        pltpu.VMEM((CAP + 16,), _i32),
        pltpu.VMEM((NPADP,), _f32),
        pltpu.VMEM((CHUNK + 16,), _f32),
        pltpu.VMEM((CHUNK, D), _f32),
        pltpu.VMEM((CHUNK, D), _f32),
        pltpu.VMEM((16,), _i32),
        pltpu.SemaphoreType.DMA,
        pltpu.SemaphoreType.DMA,
    ],
)
def _sc_fold(hw_hbm, pk_hbm, dinv_hbm, cnt_hbm, out_hbm,
             acc, sidx, dloc, dinv_v, nb, buf0, buf1, cnt_v, sem0, sem1):
    c = lax.axis_index("c")
    s = lax.axis_index("s")
    w = s * NC + c
    lo = w * RANGE

    @pl.loop(0, (RANGE + 1) * D // 16)
    def _zero(k):
        acc[pl.ds(k * 16, 16)] = jnp.zeros((16,), _f32)

    pltpu.sync_copy(pk_hbm.at[w], sidx)
    pltpu.sync_copy(dinv_hbm, dinv_v)

    @pl.loop(0, CAP // 16)
    def _unpack(k):
        pk = sidx[pl.ds(k * 16, 16)]
        dloc[pl.ds(k * 16, 16)] = jnp.minimum(pk & 511, RANGE)
        sidx[pl.ds(k * 16, 16)] = lax.shift_right_logical(pk, 9)

    pltpu.sync_copy(cnt_hbm.at[w], cnt_v)
    cnt = jnp.minimum(cnt_v[pl.ds(0, 16)][0], CAP)
    nch = jnp.maximum(((cnt + 2 * CHUNK - 1) // (2 * CHUNK)) * 2, 2)

    bufs = (buf0, buf1)
    sems = (sem0, sem1)
    pltpu.async_copy(hw_hbm.at[sidx.at[pl.ds(0, CHUNK)]], buf0, sem0)
    pltpu.async_copy(hw_hbm.at[sidx.at[pl.ds(CHUNK, CHUNK)]], buf1, sem1)

    @pl.loop(0, nch, step=2)
    def _chunks(j):
        for b in range(2):
            jj = j + b
            e0 = jj * CHUNK
            pltpu.make_async_copy(
                hw_hbm.at[sidx.at[pl.ds(e0, CHUNK)]], bufs[b], sems[b]).wait()
            for k in range(CHUNK // 16):
                sv = sidx[pl.ds(e0 + k * 16, 16)]
                dv = dloc[pl.ds(e0 + k * 16, 16)] + lo
                dsv = plsc.load_gather(dinv_v, [sv])
                ddv = plsc.load_gather(dinv_v, [dv])
                nb[pl.ds(k * 16, 16)] = dsv * ddv

            @pl.loop(0, CHUNK)
            def _fold(e):
                n = nb[pl.ds(e, 16)][0]
                base = dloc[pl.ds(e0 + e, 16)][0] * D
                for k in range(D // 16):
                    plsc.addupdate(acc.at[pl.ds(base + k * 16, 16)],
                                   bufs[b][e, pl.ds(k * 16, 16)] * n)

            @pl.when(jj + 2 < nch)
            def _next():
                pltpu.async_copy(
                    hw_hbm.at[sidx.at[pl.ds((jj + 2) * CHUNK, CHUNK)]],
                    bufs[b], sems[b])

    for t, sz in ((0, 128), (1, 128), (2, 64)):
        pltpu.sync_copy(hw_hbm.at[pl.ds(lo + t * 128, sz)],
                        buf0.at[pl.ds(0, sz)])

        @pl.loop(0, sz)
        def _selfe(e):
            di = dinv_v[pl.ds(lo + t * 128 + e, 16)][0]
            n = di * di
            base = (t * 128 + e) * D
            for k in range(D // 16):
                plsc.addupdate(acc.at[pl.ds(base + k * 16, 16)],
                               buf0[e, pl.ds(k * 16, 16)] * n)

    pltpu.sync_copy(acc.at[pl.ds(0, RANGE * D)],
                    out_hbm.at[pl.ds(lo * D, RANGE * D)])


@functools.partial(
    pl.kernel,
    out_type=jax.ShapeDtypeStruct((POOL, D), _f32),
    mesh=_mesh,
    scratch_types=[
        pltpu.VMEM_SHARED((TPOOL, D), _f32),
        pltpu.VMEM((RPS // PCH, PCH), _i32),
        pltpu.VMEM((PCH, D), _f32),
    ],
)
def _sc_pool(o_hbm, rank_hbm, out_hbm, tab_s, ridx_v, buf_v):
    c = lax.axis_index("c")
    s = lax.axis_index("s")

    @pl.when(c == 0)
    def _scatter():
        pltpu.sync_copy(rank_hbm.at[s], ridx_v)
        for j in range(RPS // PCH):
            pltpu.sync_copy(o_hbm.at[pl.ds(s * RPS + j * PCH, PCH)], buf_v)
            pltpu.sync_copy(buf_v, tab_s.at[ridx_v.at[j]])

    plsc.subcore_barrier()

    @pl.when((c == 0) & (s < 5))
    def _writeback():
        pltpu.sync_copy(tab_s.at[pl.ds(s * 200, 200)],
                        out_hbm.at[pl.ds(s * 200, 200)])


def _dot(a, b):
    return jnp.dot(a, b, preferred_element_type=_f32)


def _row_mask(shape):
    i = pl.program_id(0)
    rows = i * RB + lax.broadcasted_iota(_i32, shape, 0)
    return rows < N


_W_SPEC = pl.BlockSpec((D, D), lambda i: (0, 0))
_B_SPEC = pl.BlockSpec((1, D), lambda i: (0, 0))
_ROW_SPEC = pl.BlockSpec((RB, D), lambda i: (i, 0))
_COL_SPEC = pl.BlockSpec((RB, 1), lambda i: (i, 0))
_MAT = jax.ShapeDtypeStruct((NPAD, D), _f32)


def _prep0_body(x_ref, wg_ref, wl_ref, bl_ref, hw_ref, lin_ref):
    xb = x_ref[...]
    hw_ref[...] = _dot(xb, wg_ref[...])
    lin_ref[...] = (_dot(xb, wl_ref[...]) + bl_ref[...]) * 0.5


_tc_prep0 = pl.pallas_call(
    _prep0_body,
    grid=(NBLK,),
    in_specs=[_ROW_SPEC, _W_SPEC, _W_SPEC, _B_SPEC],
    out_specs=[_ROW_SPEC, _ROW_SPEC],
    out_shape=[_MAT, _MAT],
)


def _layer_body(agg_ref, lin_ref, bg_ref, wg_ref, wl_ref, bl_ref,
                hw_ref, lin_out_ref):
    h = jax.nn.relu((agg_ref[...] + bg_ref[...]) + lin_ref[...])
    h = jnp.where(_row_mask(h.shape), h, 0.0)
    hw_ref[...] = _dot(h, wg_ref[...])
    lin_out_ref[...] = (_dot(h, wl_ref[...]) + bl_ref[...]) * 0.5


_tc_layer = pl.pallas_call(
    _layer_body,
    grid=(NBLK,),
    in_specs=[_ROW_SPEC, _ROW_SPEC, _B_SPEC, _W_SPEC, _W_SPEC, _B_SPEC],
    out_specs=[_ROW_SPEC, _ROW_SPEC],
    out_shape=[_MAT, _MAT],
)


def _last_body(agg_ref, lin_ref, bg_ref, wg_ref, hw_ref):
    h = jax.nn.relu((agg_ref[...] + bg_ref[...]) + lin_ref[...])
    h = jnp.where(_row_mask(h.shape), h, 0.0)
    hw_ref[...] = _dot(h, wg_ref[...])


_tc_last = pl.pallas_call(
    _last_body,
    grid=(NBLK,),
    in_specs=[_ROW_SPEC, _ROW_SPEC, _B_SPEC, _W_SPEC],
    out_specs=_ROW_SPEC,
    out_shape=_MAT,
)


def _combine_body(agg_ref, bl_ref, o_ref, kcol_ref):
    o = agg_ref[...] + bl_ref[...]
    mask = _row_mask(o.shape)
    o = jnp.where(mask, o, 0.0)
    o_ref[...] = o
    kcol_ref[...] = jnp.where(mask[:, :1], o[:, D - 1:D], -3.0e38)


_tc_combine = pl.pallas_call(
    _combine_body,
    grid=(NBLK,),
    in_specs=[_ROW_SPEC, _B_SPEC],
    out_specs=[_ROW_SPEC, _COL_SPEC],
    out_shape=[_MAT, jax.ShapeDtypeStruct((NPAD, 1), _f32)],
)


def _rank_body(kcol_ref, krow_ref, rank_ref):
    i = pl.program_id(0)
    ki = kcol_ref[...]
    ig = i * IB + lax.broadcasted_iota(_i32, (IB, 1), 0)
    acc = jnp.zeros((IB, 1), _i32)
    for jb in range(NPAD // JC):
        kj = krow_ref[0, pl.ds(jb * JC, JC)][None, :]
        jg = jb * JC + lax.broadcasted_iota(_i32, (IB, JC), 1)
        gt = (kj > ki).astype(_i32)
        eq = ((kj == ki) & (jg < ig)).astype(_i32)
        acc = acc + jnp.sum(gt + eq, axis=1, keepdims=True)
    rank_ref[...] = jnp.minimum(acc, POOL)


_tc_rank = pl.pallas_call(
    _rank_body,
    grid=(NPAD // IB,),
    in_specs=[
        pl.BlockSpec((IB, 1), lambda i: (i, 0)),
        pl.BlockSpec((1, NPAD), lambda i: (0, 0)),
    ],
    out_specs=pl.BlockSpec((IB, 1), lambda i: (i, 0)),
    out_shape=jax.ShapeDtypeStruct((NPAD, 1), _i32),
)


def kernel(x, edge_index, batch, Wg0, bg0, Wl0, bl0, Wg1, bg1, Wl1, bl1,
           Wlast, blast):
    src = edge_index[0].astype(_i32)
    dst = edge_index[1].astype(_i32)
    pad_e = EPAD - E
    dstp = jnp.concatenate([dst, jnp.full((pad_e,), N, _i32)]).reshape(
        NW, NCH, CHUNK)
    xp = jnp.pad(x, ((0, NPAD - N), (0, 0)))
    zrow = jnp.zeros((RPS,), _f32)
    bg0r, bl0r, bg1r, bl1r, blr = (
        b.reshape(1, D) for b in (bg0, bl0, bg1, bl1, blast))

    deg2 = _sc_degree(dstp, zrow)
    deg = deg2[0] + deg2[1] + 1.0
    dinv = jnp.where(deg > 0, 1.0 / jnp.sqrt(deg), 0.0)
    dinvp = jnp.concatenate([dinv, jnp.ones((NPADP - NPAD,), _f32)])

    sl, cnts = _sc_prep(src, dst)
    hw0, lin0 = _tc_prep0(xp, Wg0, Wl0, bl0r)
    agg = _sc_fold(hw0, sl, dinvp, cnts).reshape(NPAD, D)
    hw1, lin1 = _tc_layer(agg, lin0, bg0r, Wg1, Wl1, bl1r)
    agg = _sc_fold(hw1, sl, dinvp, cnts).reshape(NPAD, D)
    hw2 = _tc_last(agg, lin1, bg1r, Wlast)
    agg = _sc_fold(hw2, sl, dinvp, cnts).reshape(NPAD, D)
    o, kcol = _tc_combine(agg, blr)
    krow = kcol.reshape(1, NPAD)
    rank = _tc_rank(kcol, krow)
    rank3 = rank.reshape(NS, RPS // PCH, PCH)
    out = _sc_pool(o, rank3)
    return out


# speedup vs baseline: 4.4040x; 1.1292x over previous
import functools

import jax
import jax.numpy as jnp
from jax import lax
from jax.experimental import pallas as pl
from jax.experimental.pallas import tpu as pltpu
from jax.experimental.pallas import tpu_sc as plsc

N = 10000
D = 128
POOL = 1000
E = 320000

NC = 2
NS = 16
NW = NC * NS
CHUNK = 128
EW = 10240
NCH = EW // CHUNK
EPAD = EW * NW

NPAD = 10240
RANGE = NPAD // NW
RPS = NPAD // NS
NPADP = NPAD + 512
CAP = 12288
CAPC = CAP // CHUNK
SCANC = 1600
NSCAN = E // SCANC
PCH = 128
RB = 512
NBLK = NPAD // RB
JC = 512
IB = 128
TPOOL = 1024

_f32 = jnp.float32
_i32 = jnp.int32

_mesh = plsc.VectorSubcoreMesh(core_axis_name="c", subcore_axis_name="s",
                               num_cores=NC, num_subcores=NS)


def _scalar(x):
    return x[0] if getattr(x, "ndim", 0) else x


@functools.partial(
    pl.kernel,
    out_type=jax.ShapeDtypeStruct((NC, NPAD), _f32),
    mesh=_mesh,
    scratch_types=[
        pltpu.VMEM_SHARED((NPAD,), _f32),
        pltpu.VMEM((NCH, CHUNK), _i32),
        pltpu.VMEM((CHUNK,), _f32),
    ],
)
def _sc_degree(dst_hbm, zrow_hbm, out_hbm, deg_s, idx_v, ones_v):
    c = lax.axis_index("c")
    s = lax.axis_index("s")
    w = s * NC + c
    pltpu.sync_copy(zrow_hbm, deg_s.at[pl.ds(s * RPS, RPS)])
    for k in range(CHUNK // 16):
        ones_v[pl.ds(k * 16, 16)] = jnp.ones((16,), _f32)
    pltpu.sync_copy(dst_hbm.at[w], idx_v)
    plsc.subcore_barrier()

    @pl.loop(0, NCH)
    def _chunks(j):
        pltpu.sync_copy(ones_v, deg_s.at[idx_v.at[j]], add=True)

    plsc.subcore_barrier()
    pltpu.sync_copy(deg_s.at[pl.ds(s * RPS, RPS)],
                    out_hbm.at[c, pl.ds(s * RPS, RPS)])


@functools.partial(
    pl.kernel,
    out_type=(jax.ShapeDtypeStruct((NW, CAP), _i32),
              jax.ShapeDtypeStruct((NW, 16), _i32)),
    mesh=_mesh,
    compiler_params=pltpu.CompilerParams(needs_layout_passes=False),
    scratch_types=[
        pltpu.VMEM((CAP,), _i32),
        pltpu.VMEM((SCANC,), _i32),
        pltpu.VMEM((SCANC,), _i32),
        pltpu.VMEM((16,), _i32),
    ],
)
def _sc_prep(src_hbm, dst_hbm, pk_hbm, cnt_hbm, pk_v, sbuf, dbuf, cnt_v):
    c = lax.axis_index("c")
    s = lax.axis_index("s")
    w = s * NC + c
    lo = w * RANGE
    cnt_v[pl.ds(0, 16)] = jnp.zeros((16,), _i32)

    @pl.loop(0, CAP // 16)
    def _fill(k):
        pk_v[pl.ds(k * 16, 16)] = jnp.full((16,), RANGE, _i32)

    lane = lax.iota(_i32, 16)

    def _vec(k, off):
        sv = sbuf[pl.ds(k * 16, 16)]
        dv = dbuf[pl.ds(k * 16, 16)]
        m = (dv >= lo) & (dv < lo + RANGE)
        key = jnp.where(m, lane, lane + 64)
        packed = jnp.where(m, sv * 512 + (dv - lo), RANGE)
        _, sval = plsc.sort_key_val(key, packed)

        @pl.when(off <= CAP - 16)
        def _store():
            pk_v[pl.ds(off, 16)] = sval

        return off + jnp.sum(m.astype(_i32))

    def _chunk(ci, off):
        pltpu.sync_copy(src_hbm.at[pl.ds(ci * SCANC, SCANC)], sbuf)
        pltpu.sync_copy(dst_hbm.at[pl.ds(ci * SCANC, SCANC)], dbuf)
        return pl.loop(0, SCANC // 16, init_carry=off)(_vec)

    off = pl.loop(0, NSCAN, init_carry=jnp.int32(0))(_chunk)
    cnt_v[...] = jnp.zeros((16,), _i32) + off
    pltpu.sync_copy(pk_v, pk_hbm.at[w])
    pltpu.sync_copy(cnt_v, cnt_hbm.at[w])


@functools.partial(
    pl.kernel,
    out_type=jax.ShapeDtypeStruct((NPAD * D,), _f32),
    mesh=_mesh,
    compiler_params=pltpu.CompilerParams(needs_layout_passes=False),
    scratch_types=[
        pltpu.VMEM(((RANGE + 1) * D,), _f32),
        pltpu.VMEM((CAP,), _i32),
        pltpu.VMEM((CAP + 16,), _i32),
        pltpu.VMEM((NPADP,), _f32),
        pltpu.VMEM((CHUNK + 16,), _f32),
        pltpu.VMEM((CHUNK, D), _f32),
        pltpu.VMEM((CHUNK, D), _f32),
        pltpu.VMEM((16,), _i32),
        pltpu.SemaphoreType.DMA,
        pltpu.SemaphoreType.DMA,
    ],
)
def _sc_fold(hw_hbm, pk_hbm, dinv_hbm, cnt_hbm, out_hbm,
             acc, sidx, dloc, dinv_v, nb, buf0, buf1, cnt_v, sem0, sem1):
    c = lax.axis_index("c")
    s = lax.axis_index("s")
    w = s * NC + c
    lo = w * RANGE

    @pl.loop(0, (RANGE + 1) * D // 16)
    def _zero(k):
        acc[pl.ds(k * 16, 16)] = jnp.zeros((16,), _f32)

    pltpu.sync_copy(pk_hbm.at[w], sidx)
    pltpu.sync_copy(dinv_hbm, dinv_v)

    @pl.loop(0, CAP // 16)
    def _unpack(k):
        pk = sidx[pl.ds(k * 16, 16)]
        dloc[pl.ds(k * 16, 16)] = jnp.minimum(pk & 511, RANGE)
        sidx[pl.ds(k * 16, 16)] = lax.shift_right_logical(pk, 9)

    pltpu.sync_copy(cnt_hbm.at[w], cnt_v)
    cnt = jnp.minimum(cnt_v[pl.ds(0, 16)][0], CAP)
    nch = jnp.maximum(((cnt + 2 * CHUNK - 1) // (2 * CHUNK)) * 2, 2)

    bufs = (buf0, buf1)
    sems = (sem0, sem1)
    pltpu.async_copy(hw_hbm.at[sidx.at[pl.ds(0, CHUNK)]], buf0, sem0)
    pltpu.async_copy(hw_hbm.at[sidx.at[pl.ds(CHUNK, CHUNK)]], buf1, sem1)

    @pl.loop(0, nch, step=2)
    def _chunks(j):
        for b in range(2):
            jj = j + b
            e0 = jj * CHUNK
            pltpu.make_async_copy(
                hw_hbm.at[sidx.at[pl.ds(e0, CHUNK)]], bufs[b], sems[b]).wait()
            for k in range(CHUNK // 16):
                sv = sidx[pl.ds(e0 + k * 16, 16)]
                dv = dloc[pl.ds(e0 + k * 16, 16)] + lo
                dsv = plsc.load_gather(dinv_v, [sv])
                ddv = plsc.load_gather(dinv_v, [dv])
                nb[pl.ds(k * 16, 16)] = dsv * ddv

            @pl.loop(0, CHUNK)
            def _fold(e):
                n = nb[pl.ds(e, 16)][0]
                base = dloc[pl.ds(e0 + e, 16)][0] * D
                for k in range(D // 16):
                    plsc.addupdate(acc.at[pl.ds(base + k * 16, 16)],
                                   bufs[b][e, pl.ds(k * 16, 16)] * n)

            @pl.when(jj + 2 < nch)
            def _next():
                pltpu.async_copy(
                    hw_hbm.at[sidx.at[pl.ds((jj + 2) * CHUNK, CHUNK)]],
                    bufs[b], sems[b])

    for t, sz in ((0, 128), (1, 128), (2, 64)):
        pltpu.sync_copy(hw_hbm.at[pl.ds(lo + t * 128, sz)],
                        buf0.at[pl.ds(0, sz)])

        @pl.loop(0, sz)
        def _selfe(e):
            di = dinv_v[pl.ds(lo + t * 128 + e, 16)][0]
            n = di * di
            base = (t * 128 + e) * D
            for k in range(D // 16):
                plsc.addupdate(acc.at[pl.ds(base + k * 16, 16)],
                               buf0[e, pl.ds(k * 16, 16)] * n)

    pltpu.sync_copy(acc.at[pl.ds(0, RANGE * D)],
                    out_hbm.at[pl.ds(lo * D, RANGE * D)])


@functools.partial(
    pl.kernel,
    out_type=jax.ShapeDtypeStruct((POOL, D), _f32),
    mesh=_mesh,
    scratch_types=[
        pltpu.VMEM_SHARED((TPOOL, D), _f32),
        pltpu.VMEM((RPS // PCH, PCH), _i32),
        pltpu.VMEM((PCH, D), _f32),
    ],
)
def _sc_pool(o_hbm, rank_hbm, out_hbm, tab_s, ridx_v, buf_v):
    c = lax.axis_index("c")
    s = lax.axis_index("s")

    @pl.when(c == 0)
    def _scatter():
        pltpu.sync_copy(rank_hbm.at[s], ridx_v)
        for j in range(RPS // PCH):
            pltpu.sync_copy(o_hbm.at[pl.ds(s * RPS + j * PCH, PCH)], buf_v)
            pltpu.sync_copy(buf_v, tab_s.at[ridx_v.at[j]])

    plsc.subcore_barrier()

    @pl.when((c == 0) & (s < 5))
    def _writeback():
        pltpu.sync_copy(tab_s.at[pl.ds(s * 200, 200)],
                        out_hbm.at[pl.ds(s * 200, 200)])


def _dot(a, b):
    return jnp.dot(a, b, preferred_element_type=_f32)


def _row_mask(shape):
    i = pl.program_id(0)
    rows = i * RB + lax.broadcasted_iota(_i32, shape, 0)
    return rows < N


_W_SPEC = pl.BlockSpec((D, D), lambda i: (0, 0))
_B_SPEC = pl.BlockSpec((1, D), lambda i: (0, 0))
_ROW_SPEC = pl.BlockSpec((RB, D), lambda i: (i, 0))
_COL_SPEC = pl.BlockSpec((RB, 1), lambda i: (i, 0))
_MAT = jax.ShapeDtypeStruct((NPAD, D), _f32)


def _prep0_body(x_ref, wg_ref, wl_ref, bl_ref, hw_ref, lin_ref):
    xb = x_ref[...]
    hw_ref[...] = _dot(xb, wg_ref[...])
    lin_ref[...] = (_dot(xb, wl_ref[...]) + bl_ref[...]) * 0.5


_tc_prep0 = pl.pallas_call(
    _prep0_body,
    grid=(NBLK,),
    in_specs=[_ROW_SPEC, _W_SPEC, _W_SPEC, _B_SPEC],
    out_specs=[_ROW_SPEC, _ROW_SPEC],
    out_shape=[_MAT, _MAT],
)


def _layer_body(agg_ref, lin_ref, bg_ref, wg_ref, wl_ref, bl_ref,
                hw_ref, lin_out_ref):
    h = jax.nn.relu((agg_ref[...] + bg_ref[...]) + lin_ref[...])
    h = jnp.where(_row_mask(h.shape), h, 0.0)
    hw_ref[...] = _dot(h, wg_ref[...])
    lin_out_ref[...] = (_dot(h, wl_ref[...]) + bl_ref[...]) * 0.5


_tc_layer = pl.pallas_call(
    _layer_body,
    grid=(NBLK,),
    in_specs=[_ROW_SPEC, _ROW_SPEC, _B_SPEC, _W_SPEC, _W_SPEC, _B_SPEC],
    out_specs=[_ROW_SPEC, _ROW_SPEC],
    out_shape=[_MAT, _MAT],
)


def _last_body(agg_ref, lin_ref, bg_ref, wg_ref, hw_ref):
    h = jax.nn.relu((agg_ref[...] + bg_ref[...]) + lin_ref[...])
    h = jnp.where(_row_mask(h.shape), h, 0.0)
    hw_ref[...] = _dot(h, wg_ref[...])


_tc_last = pl.pallas_call(
    _last_body,
    grid=(NBLK,),
    in_specs=[_ROW_SPEC, _ROW_SPEC, _B_SPEC, _W_SPEC],
    out_specs=_ROW_SPEC,
    out_shape=_MAT,
)


def _combine_body(agg_ref, bl_ref, o_ref, kcol_ref):
    o = agg_ref[...] + bl_ref[...]
    mask = _row_mask(o.shape)
    o = jnp.where(mask, o, 0.0)
    o_ref[...] = o
    kcol_ref[...] = jnp.where(mask[:, :1], o[:, D - 1:D], -3.0e38)


_tc_combine = pl.pallas_call(
    _combine_body,
    grid=(NBLK,),
    in_specs=[_ROW_SPEC, _B_SPEC],
    out_specs=[_ROW_SPEC, _COL_SPEC],
    out_shape=[_MAT, jax.ShapeDtypeStruct((NPAD, 1), _f32)],
)


def _rank_body(kcol_ref, krow_ref, rank_ref):
    i = pl.program_id(0)
    ki = kcol_ref[...]
    ig = i * IB + lax.broadcasted_iota(_i32, (IB, 1), 0)
    acc = jnp.zeros((IB, 1), _i32)
    for jb in range(NPAD // JC):
        kj = krow_ref[0, pl.ds(jb * JC, JC)][None, :]
        jg = jb * JC + lax.broadcasted_iota(_i32, (IB, JC), 1)
        gt = (kj > ki).astype(_i32)
        eq = ((kj == ki) & (jg < ig)).astype(_i32)
        acc = acc + jnp.sum(gt + eq, axis=1, keepdims=True)
    rank_ref[...] = jnp.minimum(acc, POOL)


_tc_rank = pl.pallas_call(
    _rank_body,
    grid=(NPAD // IB,),
    in_specs=[
        pl.BlockSpec((IB, 1), lambda i: (i, 0)),
        pl.BlockSpec((1, NPAD), lambda i: (0, 0)),
    ],
    out_specs=pl.BlockSpec((IB, 1), lambda i: (i, 0)),
    out_shape=jax.ShapeDtypeStruct((NPAD, 1), _i32),
)


def kernel(x, edge_index, batch, Wg0, bg0, Wl0, bl0, Wg1, bg1, Wl1, bl1,
           Wlast, blast):
    src = edge_index[0].astype(_i32)
    dst = edge_index[1].astype(_i32)
    pad_e = EPAD - E
    dstp = jnp.concatenate([dst, jnp.full((pad_e,), N, _i32)]).reshape(
        NW, NCH, CHUNK)
    xp = jnp.pad(x, ((0, NPAD - N), (0, 0)))
    zrow = jnp.zeros((RPS,), _f32)
    bg0r, bl0r, bg1r, bl1r, blr = (
        b.reshape(1, D) for b in (bg0, bl0, bg1, bl1, blast))

    deg2 = _sc_degree(dstp, zrow)
    deg = deg2[0] + deg2[1] + 1.0
    dinv = jnp.where(deg > 0, 1.0 / jnp.sqrt(deg), 0.0)
    dinvp = jnp.concatenate([dinv, jnp.ones((NPADP - NPAD,), _f32)])

    sl, cnts = _sc_prep(src, dst)
    hw0, lin0 = _tc_prep0(xp, Wg0, Wl0, bl0r)
    agg = _sc_fold(hw0, sl, dinvp, cnts).reshape(NPAD, D)
    hw1, lin1 = _tc_layer(agg, lin0, bg0r, Wg1, Wl1, bl1r)
    agg = _sc_fold(hw1, sl, dinvp, cnts).reshape(NPAD, D)
    hw2 = _tc_last(agg, lin1, bg1r, Wlast)
    agg = _sc_fold(hw2, sl, dinvp, cnts).reshape(NPAD, D)
    o, kcol = _tc_combine(agg, blr)
    krow = kcol.reshape(1, NPAD)
    rank = _tc_rank(kcol, krow)
    rank3 = rank.reshape(NS, RPS // PCH, PCH)
    out = _sc_pool(o, rank3)
    return out


# speedup vs baseline: 4.5263x; 1.0278x over previous
import functools

import jax
import jax.numpy as jnp
from jax import lax
from jax.experimental import pallas as pl
from jax.experimental.pallas import tpu as pltpu
from jax.experimental.pallas import tpu_sc as plsc

N = 10000
D = 128
POOL = 1000
E = 320000

NC = 2
NS = 16
NW = NC * NS
CHUNK = 128
EW = 10240
NCH = EW // CHUNK
EPAD = EW * NW

NPAD = 10240
RANGE = NPAD // NW
RPS = NPAD // NS
NPADP = NPAD + 512
CAP = 12288
CAPC = CAP // CHUNK
SCANC = 1600
NSCAN = E // SCANC
PCH = 128
RB = 512
NBLK = NPAD // RB
JC = 512
IB = 128
TPOOL = 1024

_f32 = jnp.float32
_i32 = jnp.int32

_mesh = plsc.VectorSubcoreMesh(core_axis_name="c", subcore_axis_name="s",
                               num_cores=NC, num_subcores=NS)


def _scalar(x):
    return x[0] if getattr(x, "ndim", 0) else x


@functools.partial(
    pl.kernel,
    out_type=jax.ShapeDtypeStruct((NC, NPAD), _f32),
    mesh=_mesh,
    scratch_types=[
        pltpu.VMEM_SHARED((NPAD,), _f32),
        pltpu.VMEM((NCH, CHUNK), _i32),
        pltpu.VMEM((CHUNK,), _f32),
    ],
)
def _sc_degree(dst_hbm, zrow_hbm, out_hbm, deg_s, idx_v, ones_v):
    c = lax.axis_index("c")
    s = lax.axis_index("s")
    w = s * NC + c
    pltpu.sync_copy(zrow_hbm, deg_s.at[pl.ds(s * RPS, RPS)])
    for k in range(CHUNK // 16):
        ones_v[pl.ds(k * 16, 16)] = jnp.ones((16,), _f32)
    pltpu.sync_copy(dst_hbm.at[w], idx_v)
    plsc.subcore_barrier()

    @pl.loop(0, NCH)
    def _chunks(j):
        pltpu.sync_copy(ones_v, deg_s.at[idx_v.at[j]], add=True)

    plsc.subcore_barrier()
    pltpu.sync_copy(deg_s.at[pl.ds(s * RPS, RPS)],
                    out_hbm.at[c, pl.ds(s * RPS, RPS)])


@functools.partial(
    pl.kernel,
    out_type=(jax.ShapeDtypeStruct((NW, CAP), _i32),
              jax.ShapeDtypeStruct((NW, 16), _i32)),
    mesh=_mesh,
    compiler_params=pltpu.CompilerParams(needs_layout_passes=False),
    scratch_types=[
        pltpu.VMEM((CAP,), _i32),
        pltpu.VMEM((SCANC,), _i32),
        pltpu.VMEM((SCANC,), _i32),
        pltpu.VMEM((16,), _i32),
    ],
)
def _sc_prep(src_hbm, dst_hbm, pk_hbm, cnt_hbm, pk_v, sbuf, dbuf, cnt_v):
    c = lax.axis_index("c")
    s = lax.axis_index("s")
    w = s * NC + c
    lo = w * RANGE
    cnt_v[pl.ds(0, 16)] = jnp.zeros((16,), _i32)

    @pl.loop(0, CAP // 16)
    def _fill(k):
        pk_v[pl.ds(k * 16, 16)] = jnp.full((16,), RANGE, _i32)

    lane = lax.iota(_i32, 16)

    def _vec(k, off):
        sv = sbuf[pl.ds(k * 16, 16)]
        dv = dbuf[pl.ds(k * 16, 16)]
        m = (dv >= lo) & (dv < lo + RANGE)
        key = jnp.where(m, lane, lane + 64)
        packed = jnp.where(m, sv * 512 + (dv - lo), RANGE)
        _, sval = plsc.sort_key_val(key, packed)

        @pl.when(off <= CAP - 16)
        def _store():
            pk_v[pl.ds(off, 16)] = sval

        return off + jnp.sum(m.astype(_i32))

    def _chunk(ci, off):
        pltpu.sync_copy(src_hbm.at[pl.ds(ci * SCANC, SCANC)], sbuf)
        pltpu.sync_copy(dst_hbm.at[pl.ds(ci * SCANC, SCANC)], dbuf)
        return pl.loop(0, SCANC // 16, init_carry=off)(_vec)

    off = pl.loop(0, NSCAN, init_carry=jnp.int32(0))(_chunk)
    cnt_v[...] = jnp.zeros((16,), _i32) + off
    pltpu.sync_copy(pk_v, pk_hbm.at[w])
    pltpu.sync_copy(cnt_v, cnt_hbm.at[w])


@functools.partial(
    pl.kernel,
    out_type=jax.ShapeDtypeStruct((NPAD * D,), _f32),
    mesh=_mesh,
    compiler_params=pltpu.CompilerParams(needs_layout_passes=False),
    scratch_types=[
        pltpu.VMEM(((RANGE + 1) * D,), _f32),
        pltpu.VMEM((CAP,), _i32),
        pltpu.VMEM((CAP + 16,), _i32),
        pltpu.VMEM((NPADP,), _f32),
        pltpu.VMEM((CHUNK + 16,), _f32),
        pltpu.VMEM((CHUNK, D), _f32),
        pltpu.VMEM((CHUNK, D), _f32),
        pltpu.VMEM((16,), _i32),
        pltpu.SemaphoreType.DMA,
        pltpu.SemaphoreType.DMA,
    ],
)
def _sc_fold(hw_hbm, pk_hbm, dinv_hbm, cnt_hbm, out_hbm,
             acc, sidx, dloc, dinv_v, nb, buf0, buf1, cnt_v, sem0, sem1):
    c = lax.axis_index("c")
    s = lax.axis_index("s")
    w = s * NC + c
    lo = w * RANGE

    @pl.loop(0, (RANGE + 1) * D // 16)
    def _zero(k):
        acc[pl.ds(k * 16, 16)] = jnp.zeros((16,), _f32)

    pltpu.sync_copy(pk_hbm.at[w], sidx)
    pltpu.sync_copy(dinv_hbm, dinv_v)

    @pl.loop(0, CAP // 16)
    def _unpack(k):
        pk = sidx[pl.ds(k * 16, 16)]
        dloc[pl.ds(k * 16, 16)] = jnp.minimum(pk & 511, RANGE) * D
        sidx[pl.ds(k * 16, 16)] = lax.shift_right_logical(pk, 9)

    pltpu.sync_copy(cnt_hbm.at[w], cnt_v)
    cnt = jnp.minimum(cnt_v[pl.ds(0, 16)][0], CAP)
    nch = jnp.maximum(((cnt + 2 * CHUNK - 1) // (2 * CHUNK)) * 2, 2)

    bufs = (buf0, buf1)
    sems = (sem0, sem1)
    pltpu.async_copy(hw_hbm.at[sidx.at[pl.ds(0, CHUNK)]], buf0, sem0)
    pltpu.async_copy(hw_hbm.at[sidx.at[pl.ds(CHUNK, CHUNK)]], buf1, sem1)

    @pl.loop(0, nch, step=2)
    def _chunks(j):
        for b in range(2):
            jj = j + b
            e0 = jj * CHUNK
            pltpu.make_async_copy(
                hw_hbm.at[sidx.at[pl.ds(e0, CHUNK)]], bufs[b], sems[b]).wait()
            for k in range(CHUNK // 16):
                sv = sidx[pl.ds(e0 + k * 16, 16)]
                dv = lax.shift_right_logical(
                    dloc[pl.ds(e0 + k * 16, 16)], 7) + lo
                dsv = plsc.load_gather(dinv_v, [sv])
                ddv = plsc.load_gather(dinv_v, [dv])
                nb[pl.ds(k * 16, 16)] = dsv * ddv

            @pl.loop(0, CHUNK, unroll=2)
            def _fold(e):
                n = nb[pl.ds(e, 16)][0]
                base = dloc[pl.ds(e0 + e, 16)][0]
                for k in range(D // 16):
                    plsc.addupdate(acc.at[pl.ds(base + k * 16, 16)],
                                   bufs[b][e, pl.ds(k * 16, 16)] * n)

            @pl.when(jj + 2 < nch)
            def _next():
                pltpu.async_copy(
                    hw_hbm.at[sidx.at[pl.ds((jj + 2) * CHUNK, CHUNK)]],
                    bufs[b], sems[b])

    for t, sz in ((0, 128), (1, 128), (2, 64)):
        pltpu.sync_copy(hw_hbm.at[pl.ds(lo + t * 128, sz)],
                        buf0.at[pl.ds(0, sz)])

        @pl.loop(0, sz)
        def _selfe(e):
            di = dinv_v[pl.ds(lo + t * 128 + e, 16)][0]
            n = di * di
            base = (t * 128 + e) * D
            for k in range(D // 16):
                plsc.addupdate(acc.at[pl.ds(base + k * 16, 16)],
                               buf0[e, pl.ds(k * 16, 16)] * n)

    pltpu.sync_copy(acc.at[pl.ds(0, RANGE * D)],
                    out_hbm.at[pl.ds(lo * D, RANGE * D)])


@functools.partial(
    pl.kernel,
    out_type=jax.ShapeDtypeStruct((POOL, D), _f32),
    mesh=_mesh,
    scratch_types=[
        pltpu.VMEM_SHARED((TPOOL, D), _f32),
        pltpu.VMEM((RPS // PCH, PCH), _i32),
        pltpu.VMEM((PCH, D), _f32),
    ],
)
def _sc_pool(o_hbm, rank_hbm, out_hbm, tab_s, ridx_v, buf_v):
    c = lax.axis_index("c")
    s = lax.axis_index("s")

    @pl.when(c == 0)
    def _scatter():
        pltpu.sync_copy(rank_hbm.at[s], ridx_v)
        for j in range(RPS // PCH):
            pltpu.sync_copy(o_hbm.at[pl.ds(s * RPS + j * PCH, PCH)], buf_v)
            pltpu.sync_copy(buf_v, tab_s.at[ridx_v.at[j]])

    plsc.subcore_barrier()

    @pl.when((c == 0) & (s < 5))
    def _writeback():
        pltpu.sync_copy(tab_s.at[pl.ds(s * 200, 200)],
                        out_hbm.at[pl.ds(s * 200, 200)])


def _dot(a, b):
    return jnp.dot(a, b, preferred_element_type=_f32)


def _row_mask(shape):
    i = pl.program_id(0)
    rows = i * RB + lax.broadcasted_iota(_i32, shape, 0)
    return rows < N


_W_SPEC = pl.BlockSpec((D, D), lambda i: (0, 0))
_B_SPEC = pl.BlockSpec((1, D), lambda i: (0, 0))
_ROW_SPEC = pl.BlockSpec((RB, D), lambda i: (i, 0))
_COL_SPEC = pl.BlockSpec((RB, 1), lambda i: (i, 0))
_MAT = jax.ShapeDtypeStruct((NPAD, D), _f32)


def _prep0_body(x_ref, wg_ref, wl_ref, bl_ref, hw_ref, lin_ref):
    xb = x_ref[...]
    hw_ref[...] = _dot(xb, wg_ref[...])
    lin_ref[...] = (_dot(xb, wl_ref[...]) + bl_ref[...]) * 0.5


_tc_prep0 = pl.pallas_call(
    _prep0_body,
    grid=(NBLK,),
    in_specs=[_ROW_SPEC, _W_SPEC, _W_SPEC, _B_SPEC],
    out_specs=[_ROW_SPEC, _ROW_SPEC],
    out_shape=[_MAT, _MAT],
)


def _layer_body(agg_ref, lin_ref, bg_ref, wg_ref, wl_ref, bl_ref,
                hw_ref, lin_out_ref):
    h = jax.nn.relu((agg_ref[...] + bg_ref[...]) + lin_ref[...])
    h = jnp.where(_row_mask(h.shape), h, 0.0)
    hw_ref[...] = _dot(h, wg_ref[...])
    lin_out_ref[...] = (_dot(h, wl_ref[...]) + bl_ref[...]) * 0.5


_tc_layer = pl.pallas_call(
    _layer_body,
    grid=(NBLK,),
    in_specs=[_ROW_SPEC, _ROW_SPEC, _B_SPEC, _W_SPEC, _W_SPEC, _B_SPEC],
    out_specs=[_ROW_SPEC, _ROW_SPEC],
    out_shape=[_MAT, _MAT],
)


def _last_body(agg_ref, lin_ref, bg_ref, wg_ref, hw_ref):
    h = jax.nn.relu((agg_ref[...] + bg_ref[...]) + lin_ref[...])
    h = jnp.where(_row_mask(h.shape), h, 0.0)
    hw_ref[...] = _dot(h, wg_ref[...])


_tc_last = pl.pallas_call(
    _last_body,
    grid=(NBLK,),
    in_specs=[_ROW_SPEC, _ROW_SPEC, _B_SPEC, _W_SPEC],
    out_specs=_ROW_SPEC,
    out_shape=_MAT,
)


def _combine_body(agg_ref, bl_ref, o_ref, kcol_ref):
    o = agg_ref[...] + bl_ref[...]
    mask = _row_mask(o.shape)
    o = jnp.where(mask, o, 0.0)
    o_ref[...] = o
    kcol_ref[...] = jnp.where(mask[:, :1], o[:, D - 1:D], -3.0e38)


_tc_combine = pl.pallas_call(
    _combine_body,
    grid=(NBLK,),
    in_specs=[_ROW_SPEC, _B_SPEC],
    out_specs=[_ROW_SPEC, _COL_SPEC],
    out_shape=[_MAT, jax.ShapeDtypeStruct((NPAD, 1), _f32)],
)


def _rank_body(kcol_ref, krow_ref, rank_ref):
    i = pl.program_id(0)
    ki = kcol_ref[...]
    ig = i * IB + lax.broadcasted_iota(_i32, (IB, 1), 0)
    acc = jnp.zeros((IB, 1), _i32)
    for jb in range(NPAD // JC):
        kj = krow_ref[0, pl.ds(jb * JC, JC)][None, :]
        jg = jb * JC + lax.broadcasted_iota(_i32, (IB, JC), 1)
        gt = (kj > ki).astype(_i32)
        eq = ((kj == ki) & (jg < ig)).astype(_i32)
        acc = acc + jnp.sum(gt + eq, axis=1, keepdims=True)
    rank_ref[...] = jnp.minimum(acc, POOL)


_tc_rank = pl.pallas_call(
    _rank_body,
    grid=(NPAD // IB,),
    in_specs=[
        pl.BlockSpec((IB, 1), lambda i: (i, 0)),
        pl.BlockSpec((1, NPAD), lambda i: (0, 0)),
    ],
    out_specs=pl.BlockSpec((IB, 1), lambda i: (i, 0)),
    out_shape=jax.ShapeDtypeStruct((NPAD, 1), _i32),
)


def kernel(x, edge_index, batch, Wg0, bg0, Wl0, bl0, Wg1, bg1, Wl1, bl1,
           Wlast, blast):
    src = edge_index[0].astype(_i32)
    dst = edge_index[1].astype(_i32)
    pad_e = EPAD - E
    dstp = jnp.concatenate([dst, jnp.full((pad_e,), N, _i32)]).reshape(
        NW, NCH, CHUNK)
    xp = jnp.pad(x, ((0, NPAD - N), (0, 0)))
    zrow = jnp.zeros((RPS,), _f32)
    bg0r, bl0r, bg1r, bl1r, blr = (
        b.reshape(1, D) for b in (bg0, bl0, bg1, bl1, blast))

    deg2 = _sc_degree(dstp, zrow)
    deg = deg2[0] + deg2[1] + 1.0
    dinv = jnp.where(deg > 0, 1.0 / jnp.sqrt(deg), 0.0)
    dinvp = jnp.concatenate([dinv, jnp.ones((NPADP - NPAD,), _f32)])

    sl, cnts = _sc_prep(src, dst)
    hw0, lin0 = _tc_prep0(xp, Wg0, Wl0, bl0r)
    agg = _sc_fold(hw0, sl, dinvp, cnts).reshape(NPAD, D)
    hw1, lin1 = _tc_layer(agg, lin0, bg0r, Wg1, Wl1, bl1r)
    agg = _sc_fold(hw1, sl, dinvp, cnts).reshape(NPAD, D)
    hw2 = _tc_last(agg, lin1, bg1r, Wlast)
    agg = _sc_fold(hw2, sl, dinvp, cnts).reshape(NPAD, D)
    o, kcol = _tc_combine(agg, blr)
    krow = kcol.reshape(1, NPAD)
    rank = _tc_rank(kcol, krow)
    rank3 = rank.reshape(NS, RPS // PCH, PCH)
    out = _sc_pool(o, rank3)
    return out
